# Initial kernel scaffold; baseline (speedup 1.0000x reference)
import functools
import jax
import jax.numpy as jnp
from jax import lax
from jax.experimental import pallas as pl
from jax.experimental.pallas import tpu as pltpu
from jax.experimental.pallas import tpu_sc as plsc

D = 256
Q = 1024
K = 100000
TOPK = 16
INV_SQRT_D = 0.0625
NEG = -1.0e30

KBLK = 2048
NKB = 49
K_PAD = KBLK * NKB
QBLK = 256
NQB = Q // QBLK

SC_NW = 32
QPW = Q // SC_NW
NCH = 8
CH = K_PAD // NCH
GRP = 16
NGRP = CH // (16 * GRP)
TOT = QPW * NCH


def _scores_body(q_ref, k_ref, o_ref):
    kb = pl.program_id(0)
    s = lax.dot_general(q_ref[...], k_ref[...],
                        (((1,), (1,)), ((), ())),
                        preferred_element_type=jnp.float32) * INV_SQRT_D
    col = kb * KBLK + lax.broadcasted_iota(jnp.int32, (QBLK, KBLK), 1)
    o_ref[...] = jnp.where(col < K, s, NEG)


def _scores(queries, keys):
    return pl.pallas_call(
        _scores_body,
        grid=(NKB, NQB),
        in_specs=[
            pl.BlockSpec((QBLK, D), lambda kb, qb: (qb, 0)),
            pl.BlockSpec((KBLK, D), lambda kb, qb: (kb, 0)),
        ],
        out_specs=pl.BlockSpec((QBLK, KBLK), lambda kb, qb: (qb, kb)),
        out_shape=jax.ShapeDtypeStruct((Q, K_PAD), jnp.float32),
    )(queries, keys)


def _miras_body(q_ref, bf_ref, cf_ref, df_ref, bd_ref, cd_ref, dd_ref,
                ml_ref, o_ref):
    q = q_ref[...]
    i = lax.broadcasted_iota(jnp.int32, (D, D), 0)
    j = lax.broadcasted_iota(jnp.int32, (D, D), 1)

    def tier(b, c, d2):
        wt = 0.1 * jnp.tanh(lax.dot_general(
            c, b, (((1,), (1,)), ((), ())), preferred_element_type=jnp.float32))
        wt = wt + jnp.where(i == j, d2, 0.0)
        return lax.dot_general(q, wt, (((1,), (0,)), ((), ())),
                               preferred_element_type=jnp.float32)

    v_f = tier(bf_ref[...], cf_ref[...], df_ref[...])
    v_d = tier(bd_ref[...], cd_ref[...], dd_ref[...])
    w = jax.nn.sigmoid(ml_ref[0, 0])
    o_ref[...] = w * v_f + (1.0 - w) * v_d


def _miras(queries, B_f, C_f, D_f, B_d, C_d, D_d, mix_logit):
    return pl.pallas_call(
        _miras_body,
        out_shape=jax.ShapeDtypeStruct((Q, D), jnp.float32),
    )(queries, B_f, C_f, D_f.reshape(1, D), B_d, C_d, D_d.reshape(1, D),
      mix_logit.reshape(1, 1))


@functools.partial(
    pl.kernel,
    out_type=jax.ShapeDtypeStruct((Q * D,), jnp.float32),
    mesh=plsc.VectorSubcoreMesh(core_axis_name="c", subcore_axis_name="s"),
    scratch_types=[
        pltpu.VMEM((2, CH), jnp.float32),
        pltpu.VMEM((TOPK,), jnp.int32),
        pltpu.VMEM((TOPK, D), jnp.float32),
        pltpu.VMEM((QPW * D,), jnp.float32),
        pltpu.VMEM((QPW * D,), jnp.float32),
        pltpu.SemaphoreType.DMA,
        pltpu.SemaphoreType.DMA,
    ],
)
def _sc_topk(scores_hbm, values_hbm, mem_hbm, out_hbm,
             sbuf, idx_v, rows_v, mem_v, out_v, sem, gsem):
    wid = lax.axis_index("s") * 2 + lax.axis_index("c")
    qbase = wid * QPW
    lanes = jnp.arange(16, dtype=jnp.int32)
    NEG_INIT = jnp.float32(-3.0e38)

    pltpu.sync_copy(mem_hbm.at[pl.ds(qbase * D, QPW * D)], mem_v)

    def chunk_src(t):
        qv = qbase + t // NCH
        cv = t % NCH
        return scores_hbm.at[pl.ds(qv * K_PAD + cv * CH, CH)]

    def start(t):
        pltpu.async_copy(chunk_src(t), sbuf.at[t % 2], sem)

    def wait(t):
        pltpu.make_async_copy(chunk_src(t), sbuf.at[t % 2], sem).wait()

    start(0)

    def tloop(t, carry):
        R, Ri, thr = carry
        at_start = (t % NCH) == 0
        R = jnp.where(at_start, jnp.full((16,), NEG_INIT), R)
        Ri = jnp.where(at_start, jnp.zeros((16,), jnp.int32), Ri)
        thr = jnp.where(at_start, NEG_INIT, thr)

        @pl.when(t + 1 < TOT)
        def _():
            start(t + 1)

        wait(t)
        slot = t % 2
        cbase = (t % NCH) * CH

        def grp_loop(g, carry):
            R, Ri, thr = carry
            base = g * (16 * GRP)
            m = sbuf[slot, pl.ds(base, 16)]
            for jj in range(1, GRP):
                m = jnp.maximum(m, sbuf[slot, pl.ds(base + jj * 16, 16)])
            gmax = jnp.max(m)

            def rescan(carry):
                R, Ri, thr = carry
                for jj in range(GRP):
                    v = sbuf[slot, pl.ds(base + jj * 16, 16)]
                    vmax = jnp.max(v)

                    def do_merge(cr):
                        R, Ri, _ = cr
                        gidx = (cbase + base + jj * 16) + lanes
                        cs, ci = plsc.sort_key_val(v, gidx)
                        csr = lax.rev(cs, (0,))
                        cir = lax.rev(ci, (0,))
                        keep = R >= csr
                        nR = jnp.where(keep, R, csr)
                        nI = jnp.where(keep, Ri, cir)
                        R2, Ri2 = plsc.sort_key_val(nR, nI)
                        return R2, Ri2, jnp.min(R2)

                    R, Ri, thr = lax.cond(vmax > thr, do_merge,
                                          lambda cr: cr, (R, Ri, thr))
                return R, Ri, thr

            return lax.cond(gmax > thr, rescan, lambda cr: cr, (R, Ri, thr))

        R, Ri, thr = lax.fori_loop(0, NGRP, grp_loop, (R, Ri, thr))

        @pl.when((t % NCH) == NCH - 1)
        def _():
            qi = t // NCH
            mx = jnp.max(R)
            e = jnp.exp(R - mx)
            w = e / jnp.sum(e)
            idx_v[...] = Ri
            pltpu.async_copy(values_hbm.at[idx_v], rows_v, gsem).wait()
            wjs = [jnp.sum(jnp.where(lanes == jj, w, 0.0))
                   for jj in range(TOPK)]
            for cg in range(D // 16):
                acc = mem_v[pl.ds(qi * D + cg * 16, 16)]
                for jj in range(TOPK):
                    acc = acc + wjs[jj] * rows_v[jj, pl.ds(cg * 16, 16)]
                out_v[pl.ds(qi * D + cg * 16, 16)] = acc

    lax.fori_loop(
        0, TOT, tloop,
        (jnp.full((16,), NEG_INIT), jnp.zeros((16,), jnp.int32),
         jnp.float32(-3.0e38)))

    pltpu.sync_copy(out_v, out_hbm.at[pl.ds(qbase * D, QPW * D)])


def kernel(queries, keys, values, B_f, C_f, D_f, B_d, C_d, D_d, mix_logit):
    scores = _scores(queries, keys)
    mem = _miras(queries, B_f, C_f, D_f, B_d, C_d, D_d, mix_logit)
    out = _sc_topk(scores.reshape(-1), values, mem.reshape(-1))
    return out.reshape(Q, D)


# speedup vs baseline: 1.7293x; 1.7293x over previous
import functools
import jax
import jax.numpy as jnp
from jax import lax
from jax.experimental import pallas as pl
from jax.experimental.pallas import tpu as pltpu
from jax.experimental.pallas import tpu_sc as plsc

D = 256
Q = 1024
K = 100000
TOPK = 16
INV_SQRT_D = 0.0625
NEG = -1.0e30

KBLK = 2048
NKB = 49
K_PAD = KBLK * NKB
QBLK = 256
NQB = Q // QBLK

SC_NW = 32
QPW = Q // SC_NW
NCH = 8
CH = K_PAD // NCH
GRP = 16
NGRP = CH // (16 * GRP)
TOT = QPW * NCH


def _scores_body(q_ref, k_ref, o_ref):
    kb = pl.program_id(0)
    s = lax.dot_general(q_ref[...], k_ref[...],
                        (((1,), (1,)), ((), ())),
                        preferred_element_type=jnp.float32) * INV_SQRT_D
    col = kb * KBLK + lax.broadcasted_iota(jnp.int32, (QBLK, KBLK), 1)
    o_ref[...] = jnp.where(col < K, s, NEG)


def _scores(queries, keys):
    return pl.pallas_call(
        _scores_body,
        grid=(NKB, NQB),
        in_specs=[
            pl.BlockSpec((QBLK, D), lambda kb, qb: (qb, 0)),
            pl.BlockSpec((KBLK, D), lambda kb, qb: (kb, 0)),
        ],
        out_specs=pl.BlockSpec((QBLK, KBLK), lambda kb, qb: (qb, kb)),
        out_shape=jax.ShapeDtypeStruct((Q, K_PAD), jnp.float32),
    )(queries, keys)


def _miras_body(q_ref, bf_ref, cf_ref, df_ref, bd_ref, cd_ref, dd_ref,
                ml_ref, o_ref):
    q = q_ref[...]
    i = lax.broadcasted_iota(jnp.int32, (D, D), 0)
    j = lax.broadcasted_iota(jnp.int32, (D, D), 1)

    def tier(b, c, d2):
        wt = 0.1 * jnp.tanh(lax.dot_general(
            c, b, (((1,), (1,)), ((), ())), preferred_element_type=jnp.float32))
        wt = wt + jnp.where(i == j, d2, 0.0)
        return lax.dot_general(q, wt, (((1,), (0,)), ((), ())),
                               preferred_element_type=jnp.float32)

    v_f = tier(bf_ref[...], cf_ref[...], df_ref[...])
    v_d = tier(bd_ref[...], cd_ref[...], dd_ref[...])
    w = jax.nn.sigmoid(ml_ref[0, 0])
    o_ref[...] = w * v_f + (1.0 - w) * v_d


def _miras(queries, B_f, C_f, D_f, B_d, C_d, D_d, mix_logit):
    return pl.pallas_call(
        _miras_body,
        out_shape=jax.ShapeDtypeStruct((Q, D), jnp.float32),
    )(queries, B_f, C_f, D_f.reshape(1, D), B_d, C_d, D_d.reshape(1, D),
      mix_logit.reshape(1, 1))


@functools.lru_cache(maxsize=1)
def _build_sc_topk():
    return functools.partial(
        pl.kernel,
        out_type=jax.ShapeDtypeStruct((Q * D,), jnp.float32),
        mesh=plsc.VectorSubcoreMesh(core_axis_name="c", subcore_axis_name="s"),
        scratch_types=[
            pltpu.VMEM((2, CH), jnp.float32),
            pltpu.VMEM((TOPK,), jnp.int32),
            pltpu.VMEM((TOPK, D), jnp.float32),
            pltpu.VMEM((QPW * D,), jnp.float32),
            pltpu.VMEM((QPW * D,), jnp.float32),
            pltpu.SemaphoreType.DMA,
            pltpu.SemaphoreType.DMA,
        ],
        compiler_params=pltpu.CompilerParams(needs_layout_passes=False),
    )(_sc_topk_body)


def _sc_topk_body(scores_hbm, values_hbm, mem_hbm, out_hbm,
             sbuf, idx_v, rows_v, mem_v, out_v, sem, gsem):
    wid = lax.axis_index("s") * 2 + lax.axis_index("c")
    qbase = wid * QPW
    lanes = jnp.arange(16, dtype=jnp.int32)
    NEG_INIT = jnp.float32(-3.0e38)

    pltpu.sync_copy(mem_hbm.at[pl.ds(qbase * D, QPW * D)], mem_v)

    def chunk_src(t):
        qv = qbase + t // NCH
        cv = t % NCH
        return scores_hbm.at[pl.ds(qv * K_PAD + cv * CH, CH)]

    def start(t):
        pltpu.async_copy(chunk_src(t), sbuf.at[t % 2], sem)

    def wait(t):
        pltpu.make_async_copy(chunk_src(t), sbuf.at[t % 2], sem).wait()

    start(0)

    def tloop(t, carry):
        R, Ri, thr = carry
        at_start = (t % NCH) == 0
        R = jnp.where(at_start, jnp.full((16,), NEG_INIT), R)
        Ri = jnp.where(at_start, jnp.zeros((16,), jnp.int32), Ri)
        thr = jnp.where(at_start, NEG_INIT, thr)

        @pl.when(t + 1 < TOT)
        def _():
            start(t + 1)

        wait(t)
        slot = t % 2
        cbase = (t % NCH) * CH

        def grp_loop(g, carry):
            R, Ri, thr = carry
            base = g * (16 * GRP)
            m = sbuf[slot, pl.ds(base, 16)]
            for jj in range(1, GRP):
                m = jnp.maximum(m, sbuf[slot, pl.ds(base + jj * 16, 16)])
            gmax = jnp.max(m)

            def rescan(carry):
                R, Ri, thr = carry
                for jj in range(GRP):
                    v = sbuf[slot, pl.ds(base + jj * 16, 16)]
                    vmax = jnp.max(v)

                    def do_merge(cr):
                        R, Ri, _ = cr
                        gidx = (cbase + base + jj * 16) + lanes
                        cs, ci = plsc.sort_key_val(v, gidx)
                        csr = lax.rev(cs, (0,))
                        cir = lax.rev(ci, (0,))
                        keep = R >= csr
                        nR = jnp.where(keep, R, csr)
                        nI = jnp.where(keep, Ri, cir)
                        R2, Ri2 = plsc.sort_key_val(nR, nI)
                        return R2, Ri2, jnp.min(R2)

                    R, Ri, thr = lax.cond(vmax > thr, do_merge,
                                          lambda cr: cr, (R, Ri, thr))
                return R, Ri, thr

            return lax.cond(gmax > thr, rescan, lambda cr: cr, (R, Ri, thr))

        R, Ri, thr = lax.fori_loop(0, NGRP, grp_loop, (R, Ri, thr))

        @pl.when((t % NCH) == NCH - 1)
        def _():
            qi = t // NCH
            mx = jnp.max(R)
            e = jnp.exp(R - mx)
            w = e / jnp.sum(e)
            idx_v[...] = Ri
            pltpu.async_copy(values_hbm.at[idx_v], rows_v, gsem).wait()
            wjs = [jnp.sum(jnp.where(lanes == jj, w, 0.0))
                   for jj in range(TOPK)]
            for cg in range(D // 16):
                acc = mem_v[pl.ds(qi * D + cg * 16, 16)]
                for jj in range(TOPK):
                    acc = acc + wjs[jj] * rows_v[jj, pl.ds(cg * 16, 16)]
                out_v[pl.ds(qi * D + cg * 16, 16)] = acc

        return R, Ri, thr

    lax.fori_loop(
        0, TOT, tloop,
        (jnp.full((16,), NEG_INIT), jnp.zeros((16,), jnp.int32),
         jnp.float32(-3.0e38)))

    pltpu.sync_copy(out_v, out_hbm.at[pl.ds(qbase * D, QPW * D)])


def kernel(queries, keys, values, B_f, C_f, D_f, B_d, C_d, D_d, mix_logit):
    scores = _scores(queries, keys)
    mem = _miras(queries, B_f, C_f, D_f, B_d, C_d, D_d, mix_logit)
    out = _build_sc_topk()(scores.reshape(-1), values, mem.reshape(-1))
    return out.reshape(Q, D)
